# baseline (speedup 1.0000x reference)
import functools

import jax
import jax.numpy as jnp
from jax.experimental import pallas as pl
from jax.experimental.pallas import tpu as pltpu

N, D, O, E = 2048, 768, 768, 8
BO = 128


def _moe_kernel(x_ref, wgt_ref, bg_ref, wet_ref, be_ref, out_ref,
                xb_ref, wb_ref):
    @pl.when(pl.program_id(0) == 0)
    def _gate():
        xb0 = x_ref[...].astype(jnp.bfloat16)
        xb_ref[...] = xb0
        logits = jax.lax.dot_general(
            xb0, wgt_ref[...].astype(jnp.bfloat16), (((1,), (0,)), ((), ())),
            preferred_element_type=jnp.float32,
        ) + bg_ref[...]

        iota = jax.lax.broadcasted_iota(jnp.int32, logits.shape, 1)
        big = jnp.int32(E)
        v0 = jnp.max(logits, axis=-1, keepdims=True)
        e0 = jnp.min(jnp.where(logits == v0, iota, big), axis=-1,
                     keepdims=True)
        masked = jnp.where(iota == e0, -jnp.inf, logits)
        v1 = jnp.max(masked, axis=-1, keepdims=True)
        e1 = jnp.min(jnp.where(masked == v1, iota, big), axis=-1,
                     keepdims=True)
        w0 = 1.0 / (1.0 + jnp.exp(v1 - v0))
        w1 = 1.0 - w0
        w_full = (jnp.where(iota == e0, w0, 0.0)
                  + jnp.where(iota == e1, w1, 0.0))
        wb_ref[...] = w_full.astype(jnp.bfloat16)

    xb = xb_ref[...]
    wb = wb_ref[...]

    acc = jax.lax.dot_general(
        wb, be_ref[...].astype(jnp.bfloat16), (((1,), (0,)), ((), ())),
        preferred_element_type=jnp.float32,
    )

    for e in range(E):
        xs = xb * wb[:, e][:, None]
        acc = acc + jax.lax.dot_general(
            xs, wet_ref[e][...].astype(jnp.bfloat16), (((1,), (1,)), ((), ())),
            preferred_element_type=jnp.float32,
        )
    out_ref[...] = acc


@jax.jit
def kernel(x, Wg, bg, We, be):
    wgt = Wg.T
    bg2 = bg[None, :]
    grid = (O // BO,)
    return pl.pallas_call(
        _moe_kernel,
        grid=grid,
        in_specs=[
            pl.BlockSpec((N, D), lambda j: (0, 0)),
            pl.BlockSpec((D, E), lambda j: (0, 0)),
            pl.BlockSpec((1, E), lambda j: (0, 0)),
            pl.BlockSpec((E, BO, D), lambda j: (0, j, 0)),
            pl.BlockSpec((E, BO), lambda j: (0, j)),
        ],
        out_specs=pl.BlockSpec((N, BO), lambda j: (0, j)),
        out_shape=jax.ShapeDtypeStruct((N, O), jnp.float32),
        scratch_shapes=[
            pltpu.VMEM((N, D), jnp.bfloat16),
            pltpu.VMEM((N, E), jnp.bfloat16),
        ],
    )(x, wgt, bg2, We, be)


# speedup vs baseline: 1.2994x; 1.2994x over previous
import functools

import jax
import jax.numpy as jnp
from jax.experimental import pallas as pl
from jax.experimental.pallas import tpu as pltpu

N, D, O, E = 2048, 768, 768, 8
BO = 384


def _moe_kernel(x_ref, wgt_ref, bg_ref, wet_ref, be_ref, out_ref,
                xb_ref, wb_ref):
    @pl.when(pl.program_id(0) == 0)
    def _gate():
        xb0 = x_ref[...].astype(jnp.bfloat16)
        xb_ref[...] = xb0
        logits = jax.lax.dot_general(
            xb0, wgt_ref[...].astype(jnp.bfloat16), (((1,), (0,)), ((), ())),
            preferred_element_type=jnp.float32,
        ) + bg_ref[...]

        iota = jax.lax.broadcasted_iota(jnp.int32, logits.shape, 1)
        big = jnp.int32(E)
        v0 = jnp.max(logits, axis=-1, keepdims=True)
        e0 = jnp.min(jnp.where(logits == v0, iota, big), axis=-1,
                     keepdims=True)
        masked = jnp.where(iota == e0, -jnp.inf, logits)
        v1 = jnp.max(masked, axis=-1, keepdims=True)
        e1 = jnp.min(jnp.where(masked == v1, iota, big), axis=-1,
                     keepdims=True)
        w0 = 1.0 / (1.0 + jnp.exp(v1 - v0))
        w1 = 1.0 - w0
        w_full = (jnp.where(iota == e0, w0, 0.0)
                  + jnp.where(iota == e1, w1, 0.0))
        wb_ref[...] = w_full.astype(jnp.bfloat16)

    xb = xb_ref[...]
    wb = wb_ref[...]

    acc = jax.lax.dot_general(
        wb, be_ref[...].astype(jnp.bfloat16), (((1,), (0,)), ((), ())),
        preferred_element_type=jnp.float32,
    )

    for e in range(E):
        xs = xb * wb[:, e][:, None]
        acc = acc + jax.lax.dot_general(
            xs, wet_ref[e][...].astype(jnp.bfloat16), (((1,), (1,)), ((), ())),
            preferred_element_type=jnp.float32,
        )
    out_ref[...] = acc


@jax.jit
def kernel(x, Wg, bg, We, be):
    wgt = Wg.T
    bg2 = bg[None, :]
    grid = (O // BO,)
    return pl.pallas_call(
        _moe_kernel,
        grid=grid,
        in_specs=[
            pl.BlockSpec((N, D), lambda j: (0, 0)),
            pl.BlockSpec((D, E), lambda j: (0, 0)),
            pl.BlockSpec((1, E), lambda j: (0, 0)),
            pl.BlockSpec((E, BO, D), lambda j: (0, j, 0)),
            pl.BlockSpec((E, BO), lambda j: (0, j)),
        ],
        out_specs=pl.BlockSpec((N, BO), lambda j: (0, j)),
        out_shape=jax.ShapeDtypeStruct((N, O), jnp.float32),
        scratch_shapes=[
            pltpu.VMEM((N, D), jnp.bfloat16),
            pltpu.VMEM((N, E), jnp.bfloat16),
        ],
    )(x, wgt, bg2, We, be)


# speedup vs baseline: 1.6554x; 1.2740x over previous
import functools

import jax
import jax.numpy as jnp
from jax.experimental import pallas as pl
from jax.experimental.pallas import tpu as pltpu

N, D, O, E = 2048, 768, 768, 8
BO = 256


def _moe_kernel(x_ref, wg_ref, bg_ref, wet_ref, be_ref, out_ref,
                xb_ref, wb_ref):
    @pl.when(pl.program_id(0) == 0)
    def _gate():
        xb = x_ref[...].astype(jnp.bfloat16)
        xb_ref[...] = xb
        logits = jax.lax.dot_general(
            wg_ref[...].astype(jnp.bfloat16), xb, (((1,), (1,)), ((), ())),
            preferred_element_type=jnp.float32,
        ) + bg_ref[...]

        iota = jax.lax.broadcasted_iota(jnp.int32, logits.shape, 0)
        big = jnp.int32(E)
        v0 = jnp.max(logits, axis=0, keepdims=True)
        e0 = jnp.min(jnp.where(logits == v0, iota, big), axis=0,
                     keepdims=True)
        masked = jnp.where(iota == e0, -jnp.inf, logits)
        v1 = jnp.max(masked, axis=0, keepdims=True)
        e1 = jnp.min(jnp.where(masked == v1, iota, big), axis=0,
                     keepdims=True)
        w0 = 1.0 / (1.0 + jnp.exp(v1 - v0))
        w1 = 1.0 - w0
        w_t = (jnp.where(iota == e0, w0, 0.0)
               + jnp.where(iota == e1, w1, 0.0))
        wb_ref[...] = jnp.transpose(w_t.astype(jnp.bfloat16))

    xb = xb_ref[...]
    wb = wb_ref[...]
    acc = jax.lax.dot_general(
        wb, be_ref[...].astype(jnp.bfloat16),
        (((1,), (0,)), ((), ())),
        preferred_element_type=jnp.float32,
    )

    for e in range(E):
        xs = xb * wb[:, e][:, None]
        acc = acc + jax.lax.dot_general(
            xs, wet_ref[e][...].astype(jnp.bfloat16),
            (((1,), (1,)), ((), ())),
            preferred_element_type=jnp.float32,
        )
    out_ref[...] = acc


@jax.jit
def kernel(x, Wg, bg, We, be):
    bg2 = bg[:, None]
    grid = (O // BO,)
    return pl.pallas_call(
        _moe_kernel,
        grid=grid,
        in_specs=[
            pl.BlockSpec((N, D), lambda j: (0, 0)),
            pl.BlockSpec((E, D), lambda j: (0, 0)),
            pl.BlockSpec((E, 1), lambda j: (0, 0)),
            pl.BlockSpec((E, BO, D), lambda j: (0, j, 0)),
            pl.BlockSpec((E, BO), lambda j: (0, j)),
        ],
        out_specs=pl.BlockSpec((N, BO), lambda j: (0, j)),
        out_shape=jax.ShapeDtypeStruct((N, O), jnp.float32),
        scratch_shapes=[
            pltpu.VMEM((N, D), jnp.bfloat16),
            pltpu.VMEM((N, E), jnp.bfloat16),
        ],
    )(x, Wg, bg2, We, be)


# speedup vs baseline: 1.7155x; 1.0363x over previous
import jax
import jax.numpy as jnp
from jax.experimental import pallas as pl
from jax.experimental.pallas import tpu as pltpu

N, D, O, E = 2048, 768, 768, 8
BO = 256
NH = N // 2


def _moe_kernel(x_ref, wg_ref, bg_ref, wet_ref, be_ref, out_ref,
                xb_ref, wb_ref):
    j = pl.program_id(0)
    i = pl.program_id(1)
    row = pl.ds(i * NH, NH)

    @pl.when(j == 0)
    def _gate():
        xb0 = x_ref[...].astype(jnp.bfloat16)
        xb_ref[row, :] = xb0
        logits = jax.lax.dot_general(
            wg_ref[...].astype(jnp.bfloat16), xb0, (((1,), (1,)), ((), ())),
            preferred_element_type=jnp.float32,
        ) + bg_ref[...]

        iota = jax.lax.broadcasted_iota(jnp.int32, logits.shape, 0)
        big = jnp.int32(E)
        v0 = jnp.max(logits, axis=0, keepdims=True)
        e0 = jnp.min(jnp.where(logits == v0, iota, big), axis=0,
                     keepdims=True)
        masked = jnp.where(iota == e0, -jnp.inf, logits)
        v1 = jnp.max(masked, axis=0, keepdims=True)
        e1 = jnp.min(jnp.where(masked == v1, iota, big), axis=0,
                     keepdims=True)
        w0 = 1.0 / (1.0 + jnp.exp(v1 - v0))
        w1 = 1.0 - w0
        w_t = (jnp.where(iota == e0, w0, 0.0)
               + jnp.where(iota == e1, w1, 0.0))
        wb_ref[row, :] = jnp.transpose(w_t.astype(jnp.bfloat16))

    xb = xb_ref[row, :]
    wb = wb_ref[row, :]
    acc = jax.lax.dot_general(
        wb, be_ref[...].astype(jnp.bfloat16),
        (((1,), (0,)), ((), ())),
        preferred_element_type=jnp.float32,
    )

    for e in range(E):
        xs = xb * wb[:, e][:, None]
        acc = acc + jax.lax.dot_general(
            xs, wet_ref[e][...].astype(jnp.bfloat16),
            (((1,), (1,)), ((), ())),
            preferred_element_type=jnp.float32,
        )
    out_ref[...] = acc


@jax.jit
def kernel(x, Wg, bg, We, be):
    bg2 = bg[:, None]
    grid = (O // BO, 2)
    return pl.pallas_call(
        _moe_kernel,
        grid=grid,
        in_specs=[
            pl.BlockSpec((NH, D), lambda j, i: (i, 0)),
            pl.BlockSpec((E, D), lambda j, i: (0, 0)),
            pl.BlockSpec((E, 1), lambda j, i: (0, 0)),
            pl.BlockSpec((E, BO, D), lambda j, i: (0, j, 0)),
            pl.BlockSpec((E, BO), lambda j, i: (0, j)),
        ],
        out_specs=pl.BlockSpec((NH, BO), lambda j, i: (i, j)),
        out_shape=jax.ShapeDtypeStruct((N, O), jnp.float32),
        scratch_shapes=[
            pltpu.VMEM((N, D), jnp.bfloat16),
            pltpu.VMEM((N, E), jnp.bfloat16),
        ],
    )(x, Wg, bg2, We, be)
